# Initial kernel scaffold; baseline (speedup 1.0000x reference)
import jax
import jax.numpy as jnp
from jax.experimental import pallas as pl


def kernel(x, values):
    raise NotImplementedError("write your pallas kernel here")


# speedup vs baseline: 2.0542x; 2.0542x over previous
import jax
import jax.numpy as jnp
from jax.experimental import pallas as pl
from jax.experimental.pallas import tpu as pltpu

B = 256
F = 512
BW_N = float(65536) ** (-0.2)
INV_SQRT2 = 0.7071067811865476
FB = 128
KC = 8


def _stats_kernel(x_ref, s_ref):
    x = x_ref[...]
    n = x.shape[0] * x.shape[1]
    mean = jnp.sum(x) / n
    var = jnp.sum((x - mean) ** 2) / (n - 1)
    std_all = jnp.sqrt(var)

    row = jax.lax.broadcasted_iota(jnp.int32, (B, 1), 0)
    v = x
    k = 2
    while k <= B:
        j = k // 2
        while j >= 1:
            down = jnp.roll(v, -j, axis=0)
            up = jnp.roll(v, j, axis=0)
            lower = (row & j) == 0
            partner = jnp.where(lower, down, up)
            asc = (row & k) == 0
            take_min = lower == asc
            v = jnp.where(take_min, jnp.minimum(v, partner),
                          jnp.maximum(v, partner))
            j //= 2
        k *= 2

    q25 = 0.25 * v[63:64, :] + 0.75 * v[64:65, :]
    q75 = 0.75 * v[191:192, :] + 0.25 * v[192:193, :]
    iqr = q75 - q25
    bw = 0.9 * jnp.minimum(std_all, iqr * (1.0 / 1.34)) * BW_N
    s_ref[...] = x * (INV_SQRT2 / bw)


def _kde_kernel(s_ref, o_ref):
    s = s_ref[...]
    si = s[:, None, :]

    def body(c, acc):
        sk = s_ref[pl.ds(c * KC, KC), :]
        z = si - sk[None, :, :]
        return acc + jnp.sum(jax.lax.erf(z), axis=1)

    acc = jax.lax.fori_loop(0, B // KC, body,
                            jnp.zeros((B, s.shape[1]), jnp.float32))
    o_ref[...] = 0.5 + acc * (0.5 / B)


def kernel(x, values):
    del values
    s = pl.pallas_call(
        _stats_kernel,
        out_shape=jax.ShapeDtypeStruct((B, F), jnp.float32),
    )(x)
    out = pl.pallas_call(
        _kde_kernel,
        grid=(F // FB,),
        in_specs=[pl.BlockSpec((B, FB), lambda i: (0, i))],
        out_specs=pl.BlockSpec((B, FB), lambda i: (0, i)),
        out_shape=jax.ShapeDtypeStruct((B, F), jnp.float32),
        compiler_params=pltpu.CompilerParams(
            dimension_semantics=("parallel",)),
    )(s)
    return out.reshape(x.shape)


# speedup vs baseline: 6.8057x; 3.3131x over previous
import jax
import jax.numpy as jnp
from jax.experimental import pallas as pl
from jax.experimental.pallas import tpu as pltpu

B = 256
F = 512
BW_N = float(65536) ** (-0.2)
INV_SQRT2 = 0.7071067811865476
FB = 128
IB = 128
KC = 64


def _sorted_cols(v):
    row = jax.lax.broadcasted_iota(jnp.int32, (B, 1), 0)
    k = 2
    while k <= B:
        j = k // 2
        while j >= 1:
            down = jnp.roll(v, -j, axis=0)
            up = jnp.roll(v, j, axis=0)
            lower = (row & j) == 0
            partner = jnp.where(lower, down, up)
            asc = (row & k) == 0
            take_min = lower == asc
            v = jnp.where(take_min, jnp.minimum(v, partner),
                          jnp.maximum(v, partner))
            j //= 2
        k *= 2
    return v


def _fused_kernel(xall_ref, xblk_ref, o_ref, s_scr, std_scr):
    j = pl.program_id(0)
    i = pl.program_id(1)

    @pl.when(jnp.logical_and(j == 0, i == 0))
    def _std():
        xx = xall_ref[...]
        n = xx.shape[0] * xx.shape[1]
        mean = jnp.sum(xx) / n
        std_scr[0, 0] = jnp.sqrt(jnp.sum((xx - mean) ** 2) / (n - 1))

    @pl.when(i == 0)
    def _sort_scale():
        xb = xblk_ref[...]
        v = _sorted_cols(xb)
        q25 = 0.25 * v[63:64, :] + 0.75 * v[64:65, :]
        q75 = 0.75 * v[191:192, :] + 0.25 * v[192:193, :]
        bw = 0.9 * jnp.minimum(std_scr[0, 0],
                               (q75 - q25) * (1.0 / 1.34)) * BW_N
        s_scr[...] = xb * (INV_SQRT2 / bw)

    si = s_scr[pl.ds(i * IB, IB), :]

    def body(c, _):
        chunk = s_scr[pl.ds(c * KC, KC), :]
        p0 = p1 = None
        for u in range(KC):
            sk = jax.lax.slice_in_dim(chunk, u, u + 1, axis=0)
            e = jax.lax.erf(si - sk)
            if u % 2 == 0:
                p0 = e if p0 is None else p0 + e
            else:
                p1 = e if p1 is None else p1 + e
        o_ref[...] += p0 + p1
        return 0

    o_ref[...] = jnp.zeros((IB, FB), jnp.float32)
    jax.lax.fori_loop(0, B // KC, body, 0)
    o_ref[...] = 0.5 + o_ref[...] * (0.5 / B)


def kernel(x, values):
    del values
    out = pl.pallas_call(
        _fused_kernel,
        grid=(F // FB, B // IB),
        in_specs=[
            pl.BlockSpec((B, F), lambda j, i: (0, 0)),
            pl.BlockSpec((B, FB), lambda j, i: (0, j)),
        ],
        out_specs=pl.BlockSpec((IB, FB), lambda j, i: (i, j)),
        out_shape=jax.ShapeDtypeStruct((B, F), jnp.float32),
        scratch_shapes=[
            pltpu.VMEM((B, FB), jnp.float32),
            pltpu.SMEM((1, 1), jnp.float32),
        ],
        compiler_params=pltpu.CompilerParams(
            dimension_semantics=("arbitrary", "arbitrary")),
    )(x, x)
    return out.reshape(x.shape)
